# Initial kernel scaffold; baseline (speedup 1.0000x reference)
import jax
import jax.numpy as jnp
from jax.experimental import pallas as pl
from jax.experimental.pallas import tpu as pltpu

E = 8
TOP_K = 2
NEG = -1.0e30


def _silu(v):
    return v / (1.0 + jnp.exp(-v))


def _gate(x, gw_ref, bias_ref):
    logits = jax.lax.dot_general(x, gw_ref[...], (((1,), (1,)), ((), ())),
                                 preferred_element_type=jnp.float32)
    logits = logits + bias_ref[...]
    iota = jax.lax.broadcasted_iota(jnp.int32, logits.shape, 1)
    m1 = jnp.max(logits, axis=1, keepdims=True)
    i1 = jnp.min(jnp.where(logits == m1, iota, E), axis=1, keepdims=True)
    masked = jnp.where(iota == i1, NEG, logits)
    m2 = jnp.max(masked, axis=1, keepdims=True)
    i2 = jnp.min(jnp.where(masked == m2, iota, E), axis=1, keepdims=True)
    e2 = jnp.exp(m2 - m1)
    w1 = 1.0 / (1.0 + e2)
    w2 = 1.0 - w1
    return i1, w1, i2, w2


def _dense_body(x_ref, gw_ref, bias_ref, wg_ref, wu_ref, wd_ref,
                wgs_ref, wus_ref, wds_ref, out_ref):
    j = pl.program_id(0)
    x = x_ref[...]
    i1, w1, i2, w2 = _gate(x, gw_ref, bias_ref)
    g = jax.lax.dot_general(x, wg_ref[0], (((1,), (1,)), ((), ())),
                            preferred_element_type=jnp.float32)
    u = jax.lax.dot_general(x, wu_ref[0], (((1,), (1,)), ((), ())),
                            preferred_element_type=jnp.float32)
    h = _silu(g) * u
    y = jax.lax.dot_general(h, wd_ref[0], (((1,), (1,)), ((), ())),
                            preferred_element_type=jnp.float32)
    cw = w1 * (i1 == j).astype(jnp.float32) + w2 * (i2 == j).astype(jnp.float32)
    contrib = cw * y

    @pl.when(j == 0)
    def _():
        gs = jax.lax.dot_general(x, wgs_ref[...], (((1,), (1,)), ((), ())),
                                 preferred_element_type=jnp.float32)
        us = jax.lax.dot_general(x, wus_ref[...], (((1,), (1,)), ((), ())),
                                 preferred_element_type=jnp.float32)
        hs = _silu(gs) * us
        shared = jax.lax.dot_general(hs, wds_ref[...], (((1,), (1,)), ((), ())),
                                     preferred_element_type=jnp.float32)
        out_ref[...] = shared + contrib

    @pl.when(j > 0)
    def _():
        out_ref[...] = out_ref[...] + contrib


def kernel(hidden_states, gate_weight, e_score_correction_bias, Wg, Wu, Wd,
           Wg_s, Wu_s, Wd_s):
    orig_shape = hidden_states.shape
    x = hidden_states.reshape(-1, orig_shape[-1])
    T, D = x.shape
    FF = Wg.shape[1]
    bias2 = e_score_correction_bias.reshape(1, E)

    out = pl.pallas_call(
        _dense_body,
        grid=(E,),
        in_specs=[
            pl.BlockSpec((T, D), lambda j: (0, 0)),
            pl.BlockSpec((E, D), lambda j: (0, 0)),
            pl.BlockSpec((1, E), lambda j: (0, 0)),
            pl.BlockSpec((1, FF, D), lambda j: (j, 0, 0)),
            pl.BlockSpec((1, FF, D), lambda j: (j, 0, 0)),
            pl.BlockSpec((1, D, FF), lambda j: (j, 0, 0)),
            pl.BlockSpec(Wg_s.shape, lambda j: (0, 0)),
            pl.BlockSpec(Wu_s.shape, lambda j: (0, 0)),
            pl.BlockSpec(Wd_s.shape, lambda j: (0, 0)),
        ],
        out_specs=pl.BlockSpec((T, D), lambda j: (0, 0)),
        out_shape=jax.ShapeDtypeStruct((T, D), jnp.float32),
    )(x, gate_weight, bias2, Wg, Wu, Wd, Wg_s, Wu_s, Wd_s)
    return out.reshape(orig_shape)


# speedup vs baseline: 1.7065x; 1.7065x over previous
import jax
import jax.numpy as jnp
from jax.experimental import pallas as pl
from jax.experimental.pallas import tpu as pltpu

E = 8
TOP_K = 2
NEG = -1.0e30


def _silu(v):
    return v / (1.0 + jnp.exp(-v))


def _gate(x, gw_ref, bias_ref):
    logits = jax.lax.dot_general(x, gw_ref[...], (((1,), (1,)), ((), ())),
                                 preferred_element_type=jnp.float32)
    logits = logits + bias_ref[...]
    iota = jax.lax.broadcasted_iota(jnp.int32, logits.shape, 1)
    m1 = jnp.max(logits, axis=1, keepdims=True)
    i1 = jnp.min(jnp.where(logits == m1, iota, E), axis=1, keepdims=True)
    masked = jnp.where(iota == i1, NEG, logits)
    m2 = jnp.max(masked, axis=1, keepdims=True)
    i2 = jnp.min(jnp.where(masked == m2, iota, E), axis=1, keepdims=True)
    e2 = jnp.exp(m2 - m1)
    w1 = 1.0 / (1.0 + e2)
    w2 = 1.0 - w1
    return i1, w1, i2, w2


def _dense_body(x_ref, gw_ref, bias_ref, wg_ref, wu_ref, wd_ref,
                wgs_ref, wus_ref, wds_ref, out_ref):
    j = pl.program_id(1)
    x = x_ref[...]
    i1, w1, i2, w2 = _gate(x, gw_ref, bias_ref)
    g = jax.lax.dot_general(x, wg_ref[0], (((1,), (1,)), ((), ())),
                            preferred_element_type=jnp.float32)
    u = jax.lax.dot_general(x, wu_ref[0], (((1,), (1,)), ((), ())),
                            preferred_element_type=jnp.float32)
    h = _silu(g) * u
    y = jax.lax.dot_general(h, wd_ref[0], (((1,), (1,)), ((), ())),
                            preferred_element_type=jnp.float32)
    cw = w1 * (i1 == j).astype(jnp.float32) + w2 * (i2 == j).astype(jnp.float32)
    contrib = cw * y

    @pl.when(j == 0)
    def _():
        gs = jax.lax.dot_general(x, wgs_ref[...], (((1,), (1,)), ((), ())),
                                 preferred_element_type=jnp.float32)
        us = jax.lax.dot_general(x, wus_ref[...], (((1,), (1,)), ((), ())),
                                 preferred_element_type=jnp.float32)
        hs = _silu(gs) * us
        shared = jax.lax.dot_general(hs, wds_ref[...], (((1,), (1,)), ((), ())),
                                     preferred_element_type=jnp.float32)
        out_ref[...] = shared + contrib

    @pl.when(j > 0)
    def _():
        out_ref[...] = out_ref[...] + contrib


def kernel(hidden_states, gate_weight, e_score_correction_bias, Wg, Wu, Wd,
           Wg_s, Wu_s, Wd_s):
    orig_shape = hidden_states.shape
    x = hidden_states.reshape(-1, orig_shape[-1])
    T, D = x.shape
    FF = Wg.shape[1]
    bias2 = e_score_correction_bias.reshape(1, E)

    BT = 1024
    out = pl.pallas_call(
        _dense_body,
        grid=(T // BT, E),
        in_specs=[
            pl.BlockSpec((BT, D), lambda i, j: (i, 0)),
            pl.BlockSpec((E, D), lambda i, j: (0, 0)),
            pl.BlockSpec((1, E), lambda i, j: (0, 0)),
            pl.BlockSpec((1, FF, D), lambda i, j: (j, 0, 0)),
            pl.BlockSpec((1, FF, D), lambda i, j: (j, 0, 0)),
            pl.BlockSpec((1, D, FF), lambda i, j: (j, 0, 0)),
            pl.BlockSpec(Wg_s.shape, lambda i, j: (0, 0)),
            pl.BlockSpec(Wu_s.shape, lambda i, j: (0, 0)),
            pl.BlockSpec(Wd_s.shape, lambda i, j: (0, 0)),
        ],
        out_specs=pl.BlockSpec((BT, D), lambda i, j: (i, 0)),
        out_shape=jax.ShapeDtypeStruct((T, D), jnp.float32),
    )(x, gate_weight, bias2, Wg, Wu, Wd, Wg_s, Wu_s, Wd_s)
    return out.reshape(orig_shape)
